# baseline (speedup 1.0000x reference)
import functools

import numpy as _np

import jax
import jax.numpy as jnp
from jax import lax
from jax.experimental import pallas as pl
from jax.experimental.pallas import tpu as pltpu
from jax.experimental.pallas import tpu_sc as plsc

N = 10000
R = 16
S = 10
D = 128
B = 320000

NC = 2
NS = 16
NW = NC * NS

CH = 32
CPW = 10
NP = NW * CPW * CH
GN = CH * S
GSUB = 4
GS = GN // GSUB

_pp = _np.arange(D)
_qq = _pp // 32
_rr = _pp % 32
SIGMA = 64 * (_rr // 16) + 16 * _qq + (_rr % 16)
SIGMA2 = SIGMA[SIGMA]

CT = 80
TPW = B // NW
DCHUNKS = TPW // CT

_mesh = lambda: plsc.VectorSubcoreMesh(
    core_axis_name="c", subcore_axis_name="s", num_cores=NC, num_subcores=NS)

_sc_params = lambda: pltpu.CompilerParams(needs_layout_passes=False, use_tc_tiling_on_sc=False)


def _wid():
    return lax.axis_index("s") * NC + lax.axis_index("c")


def _round_bf16_bits(u):
    return (u + 0x7FFF + ((u >> 16) & 1)) >> 16


def _pack_pair(hi, lo):
    bl = _round_bf16_bits(lax.bitcast_convert_type(lo, jnp.uint32))
    bh = _round_bf16_bits(lax.bitcast_convert_type(hi, jnp.uint32))
    return lax.bitcast_convert_type((bh << 16) | bl, jnp.int32)


def _xw_body(x_ref, wl_ref, wh_ref, ws_ref, on_ref, os_ref):
    x = x_ref[...]
    lo = jnp.dot(x, wl_ref[...], preferred_element_type=jnp.float32)
    hi = jnp.dot(x, wh_ref[...], preferred_element_type=jnp.float32)
    on_ref[...] = _pack_pair(hi, lo)
    os_ref[...] = jnp.dot(x, ws_ref[...], preferred_element_type=jnp.float32)


def _transform(h_pad, w_lo, w_hi, w_self):
    BN = 512
    return pl.pallas_call(
        _xw_body,
        grid=(NP // BN,),
        in_specs=[
            pl.BlockSpec((BN, D), lambda i: (i, 0)),
            pl.BlockSpec((D, R * D // 2), lambda i: (0, 0)),
            pl.BlockSpec((D, R * D // 2), lambda i: (0, 0)),
            pl.BlockSpec((D, D), lambda i: (0, 0)),
        ],
        out_specs=[
            pl.BlockSpec((BN, R * D // 2), lambda i: (i, 0)),
            pl.BlockSpec((BN, D), lambda i: (i, 0)),
        ],
        out_shape=[
            jax.ShapeDtypeStruct((NP, R * D // 2), jnp.int32),
            jax.ShapeDtypeStruct((NP, D), jnp.float32),
        ],
    )(h_pad, w_lo, w_hi, w_self)


def _agg_body(xwn_hbm, xws_hbm, nidx_hbm, out_hbm, nidx_all,
              rows0, rows1, self0, self1, outv0, outv1, sem0, sem1):
    rows_v = (rows0, rows1)
    self_v = (self0, self1)
    out_v = (outv0, outv1)
    sems = (sem0, sem1)
    wid = _wid()

    pltpu.sync_copy(nidx_hbm.at[pl.ds(wid * CPW * GN, CPW * GN)], nidx_all)

    def fire(slot, c):
        g = wid * CPW + c
        for j in range(GSUB):
            pltpu.async_copy(
                xwn_hbm.at[nidx_all.at[pl.ds(c * GN + j * GS, GS)]],
                rows_v[slot].at[pl.ds(j * GS, GS)], sems[slot])
        pltpu.async_copy(xws_hbm.at[pl.ds(g * CH, CH)], self_v[slot],
                         sems[slot])

    def drain(slot, c):
        g = wid * CPW + c
        for j in range(GSUB):
            pltpu.make_async_copy(
                xwn_hbm.at[nidx_all.at[pl.ds(c * GN + j * GS, GS)]],
                rows_v[slot].at[pl.ds(j * GS, GS)], sems[slot]).wait()
        pltpu.make_async_copy(xws_hbm.at[pl.ds(g * CH, CH)], self_v[slot],
                              sems[slot]).wait()

    def compute(slot, c):
        g = wid * CPW + c

        def node_body(i, _):
            base = i * S
            for q in range(D // 32):
                sl_e = pl.ds(q * 32, 16)
                sl_o = pl.ds(q * 32 + 16, 16)
                e_acc = jnp.zeros((16,), jnp.float32)
                o_acc = jnp.zeros((16,), jnp.float32)
                for s in range(S):
                    w32 = rows_v[slot][base + s, pl.ds(q * 16, 16)]
                    ev = plsc.bitcast(w32 << 16, jnp.float32)
                    ov = plsc.bitcast(w32 & jnp.int32(-65536), jnp.float32)
                    e_acc = e_acc + ev
                    o_acc = o_acc + ov
                out_v[slot][i, sl_e] = jnp.maximum(
                    self_v[slot][i, sl_e] + e_acc * (1.0 / S), 0.0)
                out_v[slot][i, sl_o] = jnp.maximum(
                    self_v[slot][i, sl_o] + o_acc * (1.0 / S), 0.0)
            return 0

        lax.fori_loop(0, CH, node_body, 0)
        pltpu.sync_copy(out_v[slot], out_hbm.at[pl.ds(g * CH, CH)])

    fire(0, 0)

    def iter_body(it, _):
        for par in range(2):
            c = it * 2 + par

            @pl.when(c + 1 < CPW)
            def _():
                fire(1 - par, c + 1)

            drain(par, c)
            compute(par, c)
        return 0

    lax.fori_loop(0, CPW // 2, iter_body, 0)


def _aggregate(xwn_packed, xws, nidx_flat):
    k = functools.partial(
        pl.kernel,
        out_type=jax.ShapeDtypeStruct((NP, D), jnp.float32),
        mesh=_mesh(),
        compiler_params=_sc_params(),
        scratch_types=[
            pltpu.VMEM((CPW * GN,), jnp.int32),
            pltpu.VMEM((GN, D // 2), jnp.int32),
            pltpu.VMEM((GN, D // 2), jnp.int32),
            pltpu.VMEM((CH, D), jnp.float32),
            pltpu.VMEM((CH, D), jnp.float32),
            pltpu.VMEM((CH, D), jnp.float32),
            pltpu.VMEM((CH, D), jnp.float32),
            pltpu.SemaphoreType.DMA,
            pltpu.SemaphoreType.DMA,
        ],
    )(_agg_body)
    return k(xwn_packed, xws, nidx_flat)


NSLOT = 5


def _dm_body(h2_hbm, rel_hbm, si_hbm, oi_hbm, ri_hbm, out_hbm, *rest):
    sr_v = rest[0:NSLOT]
    or_v = rest[NSLOT:2 * NSLOT]
    out_v = rest[2 * NSLOT:3 * NSLOT]
    sems = rest[3 * NSLOT:4 * NSLOT]
    rtab_v, si_all, oi_all, ri_all = rest[4 * NSLOT:]
    wid = _wid()
    lanes = lax.broadcasted_iota(jnp.int32, (16,), 0)

    pltpu.sync_copy(rel_hbm, rtab_v)
    pltpu.sync_copy(si_hbm.at[pl.ds(wid * TPW, TPW)], si_all)
    pltpu.sync_copy(oi_hbm.at[pl.ds(wid * TPW, TPW)], oi_all)
    pltpu.sync_copy(ri_hbm.at[pl.ds(wid * TPW, TPW)], ri_all)

    def fire(slot, sc):
        pltpu.async_copy(h2_hbm.at[si_all.at[pl.ds(sc * CT, CT)]],
                         sr_v[slot], sems[slot])
        pltpu.async_copy(h2_hbm.at[oi_all.at[pl.ds(sc * CT, CT)]],
                         or_v[slot], sems[slot])

    for k in range(NSLOT - 1):
        fire(k, k)

    def iter_body(it, _):
        for k in range(NSLOT):
            sc = it * NSLOT + k
            kn = (k + NSLOT - 1) % NSLOT

            @pl.when(sc + NSLOT - 1 < DCHUNKS)
            def _():
                fire(kn, sc + NSLOT - 1)

            pltpu.make_async_copy(h2_hbm.at[si_all.at[pl.ds(sc * CT, CT)]],
                                  sr_v[k], sems[k]).wait()
            pltpu.make_async_copy(h2_hbm.at[oi_all.at[pl.ds(sc * CT, CT)]],
                                  or_v[k], sems[k]).wait()

            def grp_body(g, _):
                scores = jnp.zeros((16,), jnp.float32)
                rel_vec = ri_all[pl.ds(sc * CT + g * 16, 16)]
                for t in range(16):
                    i = g * 16 + t
                    ri = rel_vec[t]
                    acc = jnp.zeros((16,), jnp.float32)
                    hi_mask = jnp.int32(-65536)
                    for q in range(D // 32):
                        sw = sr_v[k][i, pl.ds(q * 16, 16)]
                        ow = or_v[k][i, pl.ds(q * 16, 16)]
                        rw = rtab_v[ri, pl.ds(q * 16, 16)]
                        a0 = plsc.bitcast(sw << 16, jnp.float32)
                        b0 = plsc.bitcast(ow << 16, jnp.float32)
                        c0 = plsc.bitcast(rw << 16, jnp.float32)
                        a1 = plsc.bitcast(sw & hi_mask, jnp.float32)
                        b1 = plsc.bitcast(ow & hi_mask, jnp.float32)
                        c1 = plsc.bitcast(rw & hi_mask, jnp.float32)
                        acc = acc + a0 * b0 * c0
                        acc = acc + a1 * b1 * c1
                    scv = jnp.sum(acc)
                    scores = jnp.where(lanes == t, scv, scores)
                out_v[k][pl.ds(g * 16, 16)] = scores
                return 0

            lax.fori_loop(0, CT // 16, grp_body, 0)
            pltpu.sync_copy(out_v[k],
                            out_hbm.at[pl.ds(wid * TPW + sc * CT, CT)])
        return 0

    lax.fori_loop(0, DCHUNKS // NSLOT, iter_body, 0)


def _distmult(h2_packed, rel_packed, subj, obj, rel):
    k = functools.partial(
        pl.kernel,
        out_type=jax.ShapeDtypeStruct((B,), jnp.float32),
        mesh=_mesh(),
        compiler_params=_sc_params(),
        scratch_types=(
            [pltpu.VMEM((CT, D // 2), jnp.int32)] * NSLOT
            + [pltpu.VMEM((CT, D // 2), jnp.int32)] * NSLOT
            + [pltpu.VMEM((CT,), jnp.float32)] * NSLOT
            + [pltpu.SemaphoreType.DMA] * NSLOT
            + [pltpu.VMEM((R, D // 2), jnp.int32),
               pltpu.VMEM((TPW,), jnp.int32),
               pltpu.VMEM((TPW,), jnp.int32),
               pltpu.VMEM((TPW,), jnp.int32)]
        ),
    )(_dm_body)
    return k(h2_packed, rel_packed, subj, obj, rel)


def _layer(h_pad, W, Wself, nidx_flat):
    w_lo = jnp.transpose(W[:, :, :D // 2], (1, 0, 2)).reshape(D, R * D // 2)
    w_hi = jnp.transpose(W[:, :, D // 2:], (1, 0, 2)).reshape(D, R * D // 2)
    xwn, xws = _transform(h_pad, w_lo, w_hi, Wself)
    return _aggregate(xwn.reshape(NP * R, D // 2), xws, nidx_flat)


def kernel(x, W1, Wself1, W2, Wself2, rel_emb, neigh_idx, neigh_rel, triples):
    flat = (neigh_idx.astype(jnp.int32) * R + neigh_rel.astype(jnp.int32))
    flat = jnp.pad(flat, ((0, NP - N), (0, 0)))
    nidx_flat = flat.reshape(NP * S)

    x_pad = jnp.pad(x, ((0, NP - N), (0, 0)))
    h1 = _layer(x_pad, W1, Wself1[:, SIGMA], nidx_flat)
    h2 = _layer(h1, W2[:, SIGMA, :], Wself2[SIGMA][:, SIGMA], nidx_flat)

    subj = triples[:, 0].astype(jnp.int32)
    obj = triples[:, 1].astype(jnp.int32)
    rel = triples[:, 2].astype(jnp.int32)
    hb = _round_bf16_bits(lax.bitcast_convert_type(h2, jnp.uint32))
    h2_packed = lax.bitcast_convert_type(
        (hb[:, D // 2:] << 16) | hb[:, :D // 2], jnp.int32)
    rb = _round_bf16_bits(
        lax.bitcast_convert_type(rel_emb[:, SIGMA2], jnp.uint32))
    rb4 = rb.reshape(R, D // 32, 2, 16)
    rel_packed = lax.bitcast_convert_type(
        (rb4[:, :, 1, :] << 16) | rb4[:, :, 0, :], jnp.int32
    ).reshape(R, D // 2)
    scores = _distmult(h2_packed, rel_packed, subj, obj, rel)
    return scores.reshape(B, 1)


# speedup vs baseline: 1.0168x; 1.0168x over previous
import functools

import numpy as _np

import jax
import jax.numpy as jnp
from jax import lax
from jax.experimental import pallas as pl
from jax.experimental.pallas import tpu as pltpu
from jax.experimental.pallas import tpu_sc as plsc

N = 10000
R = 16
S = 10
D = 128
B = 320000

NC = 2
NS = 16
NW = NC * NS

CH = 32
CPW = 10
NP = NW * CPW * CH
GN = CH * S
GSUB = 4
GS = GN // GSUB

_pp = _np.arange(D)
_qq = _pp // 32
_rr = _pp % 32
SIGMA = 64 * (_rr // 16) + 16 * _qq + (_rr % 16)
SIGMA2 = SIGMA[SIGMA]

CT = 80
TPW = B // NW
DCHUNKS = TPW // CT

_mesh = lambda: plsc.VectorSubcoreMesh(
    core_axis_name="c", subcore_axis_name="s", num_cores=NC, num_subcores=NS)

_sc_params = lambda: pltpu.CompilerParams(needs_layout_passes=False, use_tc_tiling_on_sc=False)


def _wid():
    return lax.axis_index("s") * NC + lax.axis_index("c")


def _round_bf16_bits(u):
    return (u + 0x7FFF + ((u >> 16) & 1)) >> 16


def _pack_pair(hi, lo):
    bl = _round_bf16_bits(lax.bitcast_convert_type(lo, jnp.uint32))
    bh = _round_bf16_bits(lax.bitcast_convert_type(hi, jnp.uint32))
    return lax.bitcast_convert_type((bh << 16) | bl, jnp.int32)


def _xw_body(x_ref, wl_ref, wh_ref, ws_ref, on_ref, os_ref):
    x = x_ref[...]
    lo = jnp.dot(x, wl_ref[...], preferred_element_type=jnp.float32)
    hi = jnp.dot(x, wh_ref[...], preferred_element_type=jnp.float32)
    on_ref[...] = _pack_pair(hi, lo)
    os_ref[...] = jnp.dot(x, ws_ref[...], preferred_element_type=jnp.float32)


def _transform(h_pad, w_lo, w_hi, w_self):
    BN = 512
    return pl.pallas_call(
        _xw_body,
        grid=(NP // BN,),
        in_specs=[
            pl.BlockSpec((BN, D), lambda i: (i, 0)),
            pl.BlockSpec((D, R * D // 2), lambda i: (0, 0)),
            pl.BlockSpec((D, R * D // 2), lambda i: (0, 0)),
            pl.BlockSpec((D, D), lambda i: (0, 0)),
        ],
        out_specs=[
            pl.BlockSpec((BN, R * D // 2), lambda i: (i, 0)),
            pl.BlockSpec((BN, D), lambda i: (i, 0)),
        ],
        out_shape=[
            jax.ShapeDtypeStruct((NP, R * D // 2), jnp.int32),
            jax.ShapeDtypeStruct((NP, D), jnp.float32),
        ],
    )(h_pad, w_lo, w_hi, w_self)


def _agg_body(xwn_hbm, xws_hbm, nidx_hbm, out_hbm, nidx_all,
              rows0, rows1, self0, self1, out_all, sem0, sem1):
    rows_v = (rows0, rows1)
    self_v = (self0, self1)
    sems = (sem0, sem1)
    wid = _wid()

    pltpu.sync_copy(nidx_hbm.at[pl.ds(wid * CPW * GN, CPW * GN)], nidx_all)

    def fire(slot, c):
        g = wid * CPW + c
        for j in range(GSUB):
            pltpu.async_copy(
                xwn_hbm.at[nidx_all.at[pl.ds(c * GN + j * GS, GS)]],
                rows_v[slot].at[pl.ds(j * GS, GS)], sems[slot])
        pltpu.async_copy(xws_hbm.at[pl.ds(g * CH, CH)], self_v[slot],
                         sems[slot])

    def drain(slot, c):
        g = wid * CPW + c
        for j in range(GSUB):
            pltpu.make_async_copy(
                xwn_hbm.at[nidx_all.at[pl.ds(c * GN + j * GS, GS)]],
                rows_v[slot].at[pl.ds(j * GS, GS)], sems[slot]).wait()
        pltpu.make_async_copy(xws_hbm.at[pl.ds(g * CH, CH)], self_v[slot],
                              sems[slot]).wait()

    def compute(slot, c):
        def node_body(i, _):
            base = i * S
            for q in range(D // 32):
                sl_e = pl.ds(q * 32, 16)
                sl_o = pl.ds(q * 32 + 16, 16)
                e_acc = jnp.zeros((16,), jnp.float32)
                o_acc = jnp.zeros((16,), jnp.float32)
                for s in range(S):
                    w32 = rows_v[slot][base + s, pl.ds(q * 16, 16)]
                    ev = plsc.bitcast(w32 << 16, jnp.float32)
                    ov = plsc.bitcast(w32 & jnp.int32(-65536), jnp.float32)
                    e_acc = e_acc + ev
                    o_acc = o_acc + ov
                out_all[c * CH + i, sl_e] = jnp.maximum(
                    self_v[slot][i, sl_e] + e_acc * (1.0 / S), 0.0)
                out_all[c * CH + i, sl_o] = jnp.maximum(
                    self_v[slot][i, sl_o] + o_acc * (1.0 / S), 0.0)
            return 0

        lax.fori_loop(0, CH, node_body, 0)

    fire(0, 0)

    def iter_body(it, _):
        for par in range(2):
            c = it * 2 + par

            @pl.when(c + 1 < CPW)
            def _():
                fire(1 - par, c + 1)

            drain(par, c)
            compute(par, c)
        return 0

    lax.fori_loop(0, CPW // 2, iter_body, 0)
    pltpu.sync_copy(out_all, out_hbm.at[pl.ds(wid * CPW * CH, CPW * CH)])


def _aggregate(xwn_packed, xws, nidx_flat):
    k = functools.partial(
        pl.kernel,
        out_type=jax.ShapeDtypeStruct((NP, D), jnp.float32),
        mesh=_mesh(),
        compiler_params=_sc_params(),
        scratch_types=[
            pltpu.VMEM((CPW * GN,), jnp.int32),
            pltpu.VMEM((GN, D // 2), jnp.int32),
            pltpu.VMEM((GN, D // 2), jnp.int32),
            pltpu.VMEM((CH, D), jnp.float32),
            pltpu.VMEM((CH, D), jnp.float32),
            pltpu.VMEM((CPW * CH, D), jnp.float32),
            pltpu.SemaphoreType.DMA,
            pltpu.SemaphoreType.DMA,
        ],
    )(_agg_body)
    return k(xwn_packed, xws, nidx_flat)


NSLOT = 5


def _dm_body(h2_hbm, rel_hbm, si_hbm, oi_hbm, ri_hbm, out_hbm, *rest):
    sr_v = rest[0:NSLOT]
    or_v = rest[NSLOT:2 * NSLOT]
    sems = rest[2 * NSLOT:3 * NSLOT]
    rtab_v, si_all, oi_all, ri_all, out_all = rest[3 * NSLOT:]
    wid = _wid()
    lanes = lax.broadcasted_iota(jnp.int32, (16,), 0)

    pltpu.sync_copy(rel_hbm, rtab_v)
    pltpu.sync_copy(si_hbm.at[pl.ds(wid * TPW, TPW)], si_all)
    pltpu.sync_copy(oi_hbm.at[pl.ds(wid * TPW, TPW)], oi_all)
    pltpu.sync_copy(ri_hbm.at[pl.ds(wid * TPW, TPW)], ri_all)

    def fire(slot, sc):
        pltpu.async_copy(h2_hbm.at[si_all.at[pl.ds(sc * CT, CT)]],
                         sr_v[slot], sems[slot])
        pltpu.async_copy(h2_hbm.at[oi_all.at[pl.ds(sc * CT, CT)]],
                         or_v[slot], sems[slot])

    for k in range(NSLOT - 1):
        fire(k, k)

    def iter_body(it, _):
        for k in range(NSLOT):
            sc = it * NSLOT + k
            kn = (k + NSLOT - 1) % NSLOT

            @pl.when(sc + NSLOT - 1 < DCHUNKS)
            def _():
                fire(kn, sc + NSLOT - 1)

            pltpu.make_async_copy(h2_hbm.at[si_all.at[pl.ds(sc * CT, CT)]],
                                  sr_v[k], sems[k]).wait()
            pltpu.make_async_copy(h2_hbm.at[oi_all.at[pl.ds(sc * CT, CT)]],
                                  or_v[k], sems[k]).wait()

            def grp_body(g, _):
                scores = jnp.zeros((16,), jnp.float32)
                rel_vec = ri_all[pl.ds(sc * CT + g * 16, 16)]
                for t in range(16):
                    i = g * 16 + t
                    ri = rel_vec[t]
                    acc = jnp.zeros((16,), jnp.float32)
                    hi_mask = jnp.int32(-65536)
                    for q in range(D // 32):
                        sw = sr_v[k][i, pl.ds(q * 16, 16)]
                        ow = or_v[k][i, pl.ds(q * 16, 16)]
                        rw = rtab_v[ri, pl.ds(q * 16, 16)]
                        a0 = plsc.bitcast(sw << 16, jnp.float32)
                        b0 = plsc.bitcast(ow << 16, jnp.float32)
                        c0 = plsc.bitcast(rw << 16, jnp.float32)
                        a1 = plsc.bitcast(sw & hi_mask, jnp.float32)
                        b1 = plsc.bitcast(ow & hi_mask, jnp.float32)
                        c1 = plsc.bitcast(rw & hi_mask, jnp.float32)
                        acc = acc + a0 * b0 * c0
                        acc = acc + a1 * b1 * c1
                    scv = jnp.sum(acc)
                    scores = jnp.where(lanes == t, scv, scores)
                out_all[pl.ds(sc * CT + g * 16, 16)] = scores
                return 0

            lax.fori_loop(0, CT // 16, grp_body, 0)
        return 0

    lax.fori_loop(0, DCHUNKS // NSLOT, iter_body, 0)
    pltpu.sync_copy(out_all, out_hbm.at[pl.ds(wid * TPW, TPW)])


def _distmult(h2_packed, rel_packed, subj, obj, rel):
    k = functools.partial(
        pl.kernel,
        out_type=jax.ShapeDtypeStruct((B,), jnp.float32),
        mesh=_mesh(),
        compiler_params=_sc_params(),
        scratch_types=(
            [pltpu.VMEM((CT, D // 2), jnp.int32)] * NSLOT
            + [pltpu.VMEM((CT, D // 2), jnp.int32)] * NSLOT
            + [pltpu.SemaphoreType.DMA] * NSLOT
            + [pltpu.VMEM((R, D // 2), jnp.int32),
               pltpu.VMEM((TPW,), jnp.int32),
               pltpu.VMEM((TPW,), jnp.int32),
               pltpu.VMEM((TPW,), jnp.int32),
               pltpu.VMEM((TPW,), jnp.float32)]
        ),
    )(_dm_body)
    return k(h2_packed, rel_packed, subj, obj, rel)


def _layer(h_pad, W, Wself, nidx_flat):
    w_lo = jnp.transpose(W[:, :, :D // 2], (1, 0, 2)).reshape(D, R * D // 2)
    w_hi = jnp.transpose(W[:, :, D // 2:], (1, 0, 2)).reshape(D, R * D // 2)
    xwn, xws = _transform(h_pad, w_lo, w_hi, Wself)
    return _aggregate(xwn.reshape(NP * R, D // 2), xws, nidx_flat)


def kernel(x, W1, Wself1, W2, Wself2, rel_emb, neigh_idx, neigh_rel, triples):
    flat = (neigh_idx.astype(jnp.int32) * R + neigh_rel.astype(jnp.int32))
    flat = jnp.pad(flat, ((0, NP - N), (0, 0)))
    nidx_flat = flat.reshape(NP * S)

    x_pad = jnp.pad(x, ((0, NP - N), (0, 0)))
    h1 = _layer(x_pad, W1, Wself1[:, SIGMA], nidx_flat)
    h2 = _layer(h1, W2[:, SIGMA, :], Wself2[SIGMA][:, SIGMA], nidx_flat)

    subj = triples[:, 0].astype(jnp.int32)
    obj = triples[:, 1].astype(jnp.int32)
    rel = triples[:, 2].astype(jnp.int32)
    hb = _round_bf16_bits(lax.bitcast_convert_type(h2, jnp.uint32))
    h2_packed = lax.bitcast_convert_type(
        (hb[:, D // 2:] << 16) | hb[:, :D // 2], jnp.int32)
    rb = _round_bf16_bits(
        lax.bitcast_convert_type(rel_emb[:, SIGMA2], jnp.uint32))
    rb4 = rb.reshape(R, D // 32, 2, 16)
    rel_packed = lax.bitcast_convert_type(
        (rb4[:, :, 1, :] << 16) | rb4[:, :, 0, :], jnp.int32
    ).reshape(R, D // 2)
    scores = _distmult(h2_packed, rel_packed, subj, obj, rel)
    return scores.reshape(B, 1)
